# baseline (speedup 1.0000x reference)
import functools

import jax
import jax.numpy as jnp
from jax import lax
from jax.experimental import pallas as pl
from jax.experimental.pallas import tpu as pltpu
from jax.experimental.pallas import tpu_sc as plsc

N = 10000
E = 320000
D = 128
NC, NS = 2, 16
NW = NC * NS
EPT = E // NW
K = 80
NCHUNK = EPT // K
RB = 640


def _row_start(s):
    return pl.multiple_of(jnp.minimum(s * RB, N - RB), 8)

_mesh = plsc.VectorSubcoreMesh(
    core_axis_name="c", subcore_axis_name="s", num_cores=NC, num_subcores=NS
)


def _zeros16():
    return jnp.zeros((16,), jnp.float32)


@functools.partial(
    pl.kernel,
    out_type=jax.ShapeDtypeStruct((4 * N,), jnp.float32),
    mesh=_mesh,
    scratch_types=[
        pltpu.VMEM((NCHUNK, K), jnp.int32),
        pltpu.VMEM((NCHUNK, K), jnp.int32),
        pltpu.VMEM((K,), jnp.float32),
        pltpu.VMEM((RB,), jnp.float32),
        pltpu.VMEM((N,), jnp.float32),
        pltpu.VMEM_SHARED((N,), jnp.float32),
        pltpu.VMEM_SHARED((N,), jnp.float32),
        pltpu.SemaphoreType.DMA,
        pltpu.SemaphoreType.DMA,
    ],
)
def _sc_counts(i0_hbm, i1_hbm, out_hbm, i0_v, i1_v, ones_v, zb_v, stage_v,
               c0_sh, c1_sh, csem0, csem1):
    c = lax.axis_index("c")
    s = lax.axis_index("s")
    wid = c * NS + s

    for j in range(K // 16):
        ones_v[pl.ds(j * 16, 16)] = jnp.ones((16,), jnp.float32)

    @pl.loop(0, RB // 16)
    def _zf(i):
        zb_v[pl.ds(i * 16, 16)] = _zeros16()

    start = _row_start(s)
    pltpu.sync_copy(zb_v, c0_sh.at[pl.ds(start, RB)])
    pltpu.sync_copy(zb_v, c1_sh.at[pl.ds(start, RB)])
    pltpu.sync_copy(i0_hbm.at[wid], i0_v)
    pltpu.sync_copy(i1_hbm.at[wid], i1_v)

    plsc.subcore_barrier()

    @pl.loop(0, NCHUNK)
    def _body(g):
        pltpu.async_copy(ones_v, c0_sh.at[i0_v.at[g]], csem0, add=True)
        pltpu.async_copy(ones_v, c1_sh.at[i1_v.at[g]], csem1, add=True)

    pltpu.make_async_copy(out_hbm.at[pl.ds(0, N)], stage_v, csem0).wait()
    pltpu.make_async_copy(out_hbm.at[pl.ds(0, N)], stage_v, csem1).wait()

    plsc.subcore_barrier()

    @pl.when(s == 0)
    def _drain():
        pltpu.sync_copy(c0_sh, stage_v)
        pltpu.sync_copy(stage_v, out_hbm.at[pl.ds((2 * c) * N, N)])
        pltpu.sync_copy(c1_sh, stage_v)
        pltpu.sync_copy(stage_v, out_hbm.at[pl.ds((2 * c + 1) * N, N)])


@functools.partial(
    pl.kernel,
    out_type=jax.ShapeDtypeStruct((2 * N, D), jnp.float32),
    mesh=_mesh,
    scratch_types=[
        pltpu.VMEM((NCHUNK, K), jnp.int32),
        pltpu.VMEM((K,), jnp.int32),
        pltpu.VMEM((K,), jnp.int32),
        pltpu.VMEM((K, D), jnp.float32),
        pltpu.VMEM((K, D), jnp.float32),
        pltpu.VMEM_SHARED((N, D), jnp.float32),
        pltpu.SemaphoreType.DMA,
        pltpu.SemaphoreType.DMA,
        pltpu.SemaphoreType.DMA,
        pltpu.SemaphoreType.DMA,
    ],
)
def _sc_seg(table_hbm, gi_hbm, si_hbm, out_hbm, si_v, gi0_v, gi1_v,
            rows0_v, rows1_v, acc_sh, sem0, sem1, isem0, isem1):
    c = lax.axis_index("c")
    s = lax.axis_index("s")
    wid = c * NS + s

    @pl.loop(0, K)
    def _zf(i):
        for j in range(D // 16):
            rows0_v[i, pl.ds(j * 16, 16)] = _zeros16()

    start = _row_start(s)
    for t in range(8):
        pltpu.sync_copy(rows0_v, acc_sh.at[pl.ds(start + t * K, K)])

    pltpu.sync_copy(si_hbm.at[wid], si_v)

    plsc.subcore_barrier()

    pltpu.sync_copy(gi_hbm.at[pl.ds(wid * EPT, K)], gi0_v)
    pltpu.sync_copy(gi_hbm.at[pl.ds(wid * EPT + K, K)], gi1_v)
    pltpu.async_copy(table_hbm.at[gi0_v], rows0_v, sem0)
    pltpu.async_copy(table_hbm.at[gi1_v], rows1_v, sem1)

    @pl.loop(0, NCHUNK - 1, step=2)
    def _body(g):
        base = wid * EPT + g * K
        pltpu.make_async_copy(table_hbm.at[gi0_v], rows0_v, sem0).wait()
        pltpu.async_copy(gi_hbm.at[pl.ds(base + 2 * K, K)], gi0_v, isem0)
        pltpu.sync_copy(rows0_v, acc_sh.at[si_v.at[g]], add=True)
        pltpu.make_async_copy(gi_hbm.at[pl.ds(base, K)], gi0_v, isem0).wait()
        pltpu.async_copy(table_hbm.at[gi0_v], rows0_v, sem0)
        pltpu.make_async_copy(table_hbm.at[gi1_v], rows1_v, sem1).wait()

        @pl.when(g + 3 < NCHUNK)
        def _pf():
            pltpu.async_copy(gi_hbm.at[pl.ds(base + 3 * K, K)], gi1_v, isem1)

        pltpu.sync_copy(rows1_v, acc_sh.at[si_v.at[g + 1]], add=True)

        @pl.when(g + 3 < NCHUNK)
        def _g3():
            pltpu.make_async_copy(gi_hbm.at[pl.ds(base, K)], gi1_v, isem1).wait()
            pltpu.async_copy(table_hbm.at[gi1_v], rows1_v, sem1)

    pltpu.make_async_copy(table_hbm.at[gi0_v], rows0_v, sem0).wait()
    pltpu.sync_copy(rows0_v, acc_sh.at[si_v.at[NCHUNK - 1]], add=True)

    plsc.subcore_barrier()
    pltpu.sync_copy(
        acc_sh.at[pl.ds(start, RB)],
        out_hbm.at[pl.ds(c * N + start, RB)],
    )


BLK = 1000


def _mm_body(x_ref, w_ref, o_ref):
    o_ref[...] = jnp.dot(x_ref[...], w_ref[...], preferred_element_type=jnp.float32)


def _tc_mm(x, w):
    return pl.pallas_call(
        _mm_body,
        grid=(N // BLK,),
        in_specs=[
            pl.BlockSpec((BLK, D), lambda i: (i, 0)),
            pl.BlockSpec((D, D), lambda i: (0, 0)),
        ],
        out_specs=pl.BlockSpec((BLK, D), lambda i: (i, 0)),
        out_shape=jax.ShapeDtypeStruct((N, D), jnp.float32),
    )(x, w)


def _recip_block(c_ref):
    tot = c_ref[:, 0:1] + c_ref[:, 1:2]
    return jnp.where(tot > 0, 1.0 / tot, 0.0)


def _comb_e_body(p_ref, r_ref, o_ref):
    o_ref[...] = (p_ref[0] + p_ref[1]) * _recip_block(r_ref)


def _tc_comb_e(p, r):
    return pl.pallas_call(
        _comb_e_body,
        grid=(N // BLK,),
        in_specs=[
            pl.BlockSpec((2, BLK, D), lambda i: (0, i, 0)),
            pl.BlockSpec((BLK, 2), lambda i: (i, 0)),
        ],
        out_specs=pl.BlockSpec((BLK, D), lambda i: (i, 0)),
        out_shape=jax.ShapeDtypeStruct((N, D), jnp.float32),
    )(p, r)


def _hmm_body(q_ref, r_ref, b_ref, w_ref, o_ref):
    h = (q_ref[0] + q_ref[1]) * _recip_block(r_ref) + b_ref[...]
    h = jnp.where(h > 0, h, jnp.exp(h) - 1.0)
    o_ref[...] = jnp.dot(h, w_ref[...], preferred_element_type=jnp.float32)


def _tc_hmm(q, r, b, w):
    return pl.pallas_call(
        _hmm_body,
        grid=(N // BLK,),
        in_specs=[
            pl.BlockSpec((2, BLK, D), lambda i: (0, i, 0)),
            pl.BlockSpec((BLK, 2), lambda i: (i, 0)),
            pl.BlockSpec((1, D), lambda i: (0, 0)),
            pl.BlockSpec((D, D), lambda i: (0, 0)),
        ],
        out_specs=pl.BlockSpec((BLK, D), lambda i: (i, 0)),
        out_shape=jax.ShapeDtypeStruct((N, D), jnp.float32),
    )(q, r, b, w)


def _fin_body(q_ref, r_ref, b_ref, o_ref):
    o_ref[...] = (q_ref[0] + q_ref[1]) * _recip_block(r_ref) + b_ref[...]


def _tc_final(q, r, b):
    return pl.pallas_call(
        _fin_body,
        grid=(N // BLK,),
        in_specs=[
            pl.BlockSpec((2, BLK, D), lambda i: (0, i, 0)),
            pl.BlockSpec((BLK, 2), lambda i: (i, 0)),
            pl.BlockSpec((1, D), lambda i: (0, 0)),
        ],
        out_specs=pl.BlockSpec((BLK, D), lambda i: (i, 0)),
        out_shape=jax.ShapeDtypeStruct((N, D), jnp.float32),
    )(q, r, b)


def kernel(x, hyperedge_index, W1, b1, W2, b2):
    idx0 = hyperedge_index[0]
    idx1 = hyperedge_index[1]
    idx0_c = idx0.reshape(NW, NCHUNK, K)
    idx1_c = idx1.reshape(NW, NCHUNK, K)

    cnts = _sc_counts(idx0_c, idx1_c).reshape(2, 2, N)
    cD = cnts[:, 0, :].T
    cB = cnts[:, 1, :].T

    b1r = b1.reshape(1, D)
    b2r = b2.reshape(1, D)

    xw1 = _tc_mm(x, W1)
    p1 = _sc_seg(xw1, idx0, idx1_c).reshape(2, N, D)
    e1 = _tc_comb_e(p1, cB)
    q1 = _sc_seg(e1, idx1, idx0_c).reshape(2, N, D)
    xw2 = _tc_hmm(q1, cD, b1r, W2)
    p2 = _sc_seg(xw2, idx0, idx1_c).reshape(2, N, D)
    e2 = _tc_comb_e(p2, cB)
    q2 = _sc_seg(e2, idx1, idx0_c).reshape(2, N, D)
    out = _tc_final(q2, cD, b2r)
    return (out, e2)


# speedup vs baseline: 1.0060x; 1.0060x over previous
import functools

import jax
import jax.numpy as jnp
from jax import lax
from jax.experimental import pallas as pl
from jax.experimental.pallas import tpu as pltpu
from jax.experimental.pallas import tpu_sc as plsc

N = 10000
E = 320000
D = 128
NC, NS = 2, 16
NW = NC * NS
EPT = E // NW
K = 80
NCHUNK = EPT // K
RB = 640


def _row_start(s):
    return pl.multiple_of(jnp.minimum(s * RB, N - RB), 8)

_mesh = plsc.VectorSubcoreMesh(
    core_axis_name="c", subcore_axis_name="s", num_cores=NC, num_subcores=NS
)


def _zeros16():
    return jnp.zeros((16,), jnp.float32)


@functools.partial(
    pl.kernel,
    out_type=jax.ShapeDtypeStruct((4 * N,), jnp.float32),
    mesh=_mesh,
    scratch_types=[
        pltpu.VMEM((NCHUNK, K), jnp.int32),
        pltpu.VMEM((NCHUNK, K), jnp.int32),
        pltpu.VMEM((K,), jnp.float32),
        pltpu.VMEM((RB,), jnp.float32),
        pltpu.VMEM((N,), jnp.float32),
        pltpu.VMEM_SHARED((N,), jnp.float32),
        pltpu.VMEM_SHARED((N,), jnp.float32),
        pltpu.SemaphoreType.DMA,
        pltpu.SemaphoreType.DMA,
    ],
)
def _sc_counts(i0_hbm, i1_hbm, out_hbm, i0_v, i1_v, ones_v, zb_v, stage_v,
               c0_sh, c1_sh, csem0, csem1):
    c = lax.axis_index("c")
    s = lax.axis_index("s")
    wid = c * NS + s

    for j in range(K // 16):
        ones_v[pl.ds(j * 16, 16)] = jnp.ones((16,), jnp.float32)

    @pl.loop(0, RB // 16)
    def _zf(i):
        zb_v[pl.ds(i * 16, 16)] = _zeros16()

    start = _row_start(s)
    pltpu.sync_copy(zb_v, c0_sh.at[pl.ds(start, RB)])
    pltpu.sync_copy(zb_v, c1_sh.at[pl.ds(start, RB)])
    pltpu.sync_copy(i0_hbm.at[wid], i0_v)
    pltpu.sync_copy(i1_hbm.at[wid], i1_v)

    plsc.subcore_barrier()

    @pl.loop(0, NCHUNK)
    def _body(g):
        pltpu.async_copy(ones_v, c0_sh.at[i0_v.at[g]], csem0, add=True)
        pltpu.async_copy(ones_v, c1_sh.at[i1_v.at[g]], csem1, add=True)

    pltpu.make_async_copy(out_hbm.at[pl.ds(0, N)], stage_v, csem0).wait()
    pltpu.make_async_copy(out_hbm.at[pl.ds(0, N)], stage_v, csem1).wait()

    plsc.subcore_barrier()

    @pl.when(s == 0)
    def _drain():
        pltpu.sync_copy(c0_sh, stage_v)
        pltpu.sync_copy(stage_v, out_hbm.at[pl.ds((2 * c) * N, N)])
        pltpu.sync_copy(c1_sh, stage_v)
        pltpu.sync_copy(stage_v, out_hbm.at[pl.ds((2 * c + 1) * N, N)])


@functools.partial(
    pl.kernel,
    out_type=jax.ShapeDtypeStruct((2 * N, D), jnp.float32),
    mesh=_mesh,
    scratch_types=[
        pltpu.VMEM((NCHUNK, K), jnp.int32),
        pltpu.VMEM((K,), jnp.int32),
        pltpu.VMEM((K,), jnp.int32),
        pltpu.VMEM((K, D), jnp.float32),
        pltpu.VMEM((K, D), jnp.float32),
        pltpu.VMEM_SHARED((N, D), jnp.float32),
        pltpu.SemaphoreType.DMA,
        pltpu.SemaphoreType.DMA,
        pltpu.SemaphoreType.DMA,
        pltpu.SemaphoreType.DMA,
    ],
)
def _sc_seg(table_hbm, gi_hbm, si_hbm, out_hbm, si_v, gi0_v, gi1_v,
            rows0_v, rows1_v, acc_sh, sem0, sem1, isem0, isem1):
    c = lax.axis_index("c")
    s = lax.axis_index("s")
    wid = c * NS + s

    @pl.loop(0, K)
    def _zf(i):
        for j in range(D // 16):
            rows0_v[i, pl.ds(j * 16, 16)] = _zeros16()

    start = _row_start(s)
    for t in range(8):
        pltpu.sync_copy(rows0_v, acc_sh.at[pl.ds(start + t * K, K)])

    pltpu.sync_copy(si_hbm.at[wid], si_v)

    plsc.subcore_barrier()

    pltpu.sync_copy(gi_hbm.at[pl.ds(wid * EPT, K)], gi0_v)
    pltpu.sync_copy(gi_hbm.at[pl.ds(wid * EPT + K, K)], gi1_v)
    pltpu.async_copy(table_hbm.at[gi0_v], rows0_v, sem0)
    pltpu.async_copy(table_hbm.at[gi1_v], rows1_v, sem1)

    @pl.loop(0, NCHUNK - 1, step=2)
    def _body(g):
        base = wid * EPT + g * K
        pltpu.make_async_copy(table_hbm.at[gi0_v], rows0_v, sem0).wait()
        pltpu.async_copy(gi_hbm.at[pl.ds(base + 2 * K, K)], gi0_v, isem0)
        pltpu.sync_copy(rows0_v, acc_sh.at[si_v.at[g]], add=True)
        pltpu.make_async_copy(gi_hbm.at[pl.ds(base, K)], gi0_v, isem0).wait()
        pltpu.async_copy(table_hbm.at[gi0_v], rows0_v, sem0)
        pltpu.make_async_copy(table_hbm.at[gi1_v], rows1_v, sem1).wait()

        @pl.when(g + 3 < NCHUNK)
        def _pf():
            pltpu.async_copy(gi_hbm.at[pl.ds(base + 3 * K, K)], gi1_v, isem1)

        pltpu.sync_copy(rows1_v, acc_sh.at[si_v.at[g + 1]], add=True)

        @pl.when(g + 3 < NCHUNK)
        def _g3():
            pltpu.make_async_copy(gi_hbm.at[pl.ds(base, K)], gi1_v, isem1).wait()
            pltpu.async_copy(table_hbm.at[gi1_v], rows1_v, sem1)

    pltpu.make_async_copy(table_hbm.at[gi0_v], rows0_v, sem0).wait()
    pltpu.sync_copy(rows0_v, acc_sh.at[si_v.at[NCHUNK - 1]], add=True)

    plsc.subcore_barrier()
    pltpu.sync_copy(
        acc_sh.at[pl.ds(start, RB)],
        out_hbm.at[pl.ds(c * N + start, RB)],
    )


BLK = 1000


def _mm_body(x_ref, w_ref, o_ref):
    o_ref[...] = jnp.dot(x_ref[...], w_ref[...], preferred_element_type=jnp.float32)


def _tc_mm(x, w):
    return pl.pallas_call(
        _mm_body,
        grid=(N // BLK,),
        in_specs=[
            pl.BlockSpec((BLK, D), lambda i: (i, 0)),
            pl.BlockSpec((D, D), lambda i: (0, 0)),
        ],
        out_specs=pl.BlockSpec((BLK, D), lambda i: (i, 0)),
        out_shape=jax.ShapeDtypeStruct((N, D), jnp.float32),
    )(x, w)


def _recips_body(c_ref, o_ref):
    tot = c_ref[0:1, :] + c_ref[1:2, :]
    o_ref[...] = jnp.where(tot > 0, 1.0 / tot, 0.0)


def _tc_recips(cnts):
    return pl.pallas_call(
        _recips_body,
        out_shape=jax.ShapeDtypeStruct((1, 2 * N), jnp.float32),
    )(cnts)


def _comb_e_body(p_ref, r_ref, o_ref):
    o_ref[...] = (p_ref[0] + p_ref[1]) * r_ref[...]


def _tc_comb_e(p, r):
    return pl.pallas_call(
        _comb_e_body,
        grid=(N // BLK,),
        in_specs=[
            pl.BlockSpec((2, BLK, D), lambda i: (0, i, 0)),
            pl.BlockSpec((BLK, 1), lambda i: (i, 0)),
        ],
        out_specs=pl.BlockSpec((BLK, D), lambda i: (i, 0)),
        out_shape=jax.ShapeDtypeStruct((N, D), jnp.float32),
    )(p, r)


def _hmm_body(q_ref, r_ref, b_ref, w_ref, o_ref):
    h = (q_ref[0] + q_ref[1]) * r_ref[...] + b_ref[...]
    h = jnp.where(h > 0, h, jnp.exp(h) - 1.0)
    o_ref[...] = jnp.dot(h, w_ref[...], preferred_element_type=jnp.float32)


def _tc_hmm(q, r, b, w):
    return pl.pallas_call(
        _hmm_body,
        grid=(N // BLK,),
        in_specs=[
            pl.BlockSpec((2, BLK, D), lambda i: (0, i, 0)),
            pl.BlockSpec((BLK, 1), lambda i: (i, 0)),
            pl.BlockSpec((1, D), lambda i: (0, 0)),
            pl.BlockSpec((D, D), lambda i: (0, 0)),
        ],
        out_specs=pl.BlockSpec((BLK, D), lambda i: (i, 0)),
        out_shape=jax.ShapeDtypeStruct((N, D), jnp.float32),
    )(q, r, b, w)


def _fin_body(q_ref, r_ref, b_ref, o_ref):
    o_ref[...] = (q_ref[0] + q_ref[1]) * r_ref[...] + b_ref[...]


def _tc_final(q, r, b):
    return pl.pallas_call(
        _fin_body,
        grid=(N // BLK,),
        in_specs=[
            pl.BlockSpec((2, BLK, D), lambda i: (0, i, 0)),
            pl.BlockSpec((BLK, 1), lambda i: (i, 0)),
            pl.BlockSpec((1, D), lambda i: (0, 0)),
        ],
        out_specs=pl.BlockSpec((BLK, D), lambda i: (i, 0)),
        out_shape=jax.ShapeDtypeStruct((N, D), jnp.float32),
    )(q, r, b)


def kernel(x, hyperedge_index, W1, b1, W2, b2):
    idx0 = hyperedge_index[0]
    idx1 = hyperedge_index[1]
    idx0_c = idx0.reshape(NW, NCHUNK, K)
    idx1_c = idx1.reshape(NW, NCHUNK, K)

    cnts = _sc_counts(idx0_c, idx1_c).reshape(2, 2 * N)
    recips = _tc_recips(cnts).reshape(2, N)
    rD = recips[0].reshape(N, 1)
    rB = recips[1].reshape(N, 1)

    b1r = b1.reshape(1, D)
    b2r = b2.reshape(1, D)

    xw1 = _tc_mm(x, W1)
    p1 = _sc_seg(xw1, idx0, idx1_c).reshape(2, N, D)
    e1 = _tc_comb_e(p1, rB)
    q1 = _sc_seg(e1, idx1, idx0_c).reshape(2, N, D)
    xw2 = _tc_hmm(q1, rD, b1r, W2)
    p2 = _sc_seg(xw2, idx0, idx1_c).reshape(2, N, D)
    e2 = _tc_comb_e(p2, rB)
    q2 = _sc_seg(e2, idx1, idx0_c).reshape(2, N, D)
    out = _tc_final(q2, rD, b2r)
    return (out, e2)


# speedup vs baseline: 1.0348x; 1.0287x over previous
import functools

import jax
import jax.numpy as jnp
from jax import lax
from jax.experimental import pallas as pl
from jax.experimental.pallas import tpu as pltpu
from jax.experimental.pallas import tpu_sc as plsc

N = 10000
E = 320000
D = 128
NC, NS = 2, 16
NW = NC * NS
EPT = E // NW
K = 80
NCHUNK = EPT // K
RB = 640


def _row_start(s):
    return pl.multiple_of(jnp.minimum(s * RB, N - RB), 8)

_mesh = plsc.VectorSubcoreMesh(
    core_axis_name="c", subcore_axis_name="s", num_cores=NC, num_subcores=NS
)


def _zeros16():
    return jnp.zeros((16,), jnp.float32)


@functools.partial(
    pl.kernel,
    out_type=jax.ShapeDtypeStruct((4 * N,), jnp.float32),
    mesh=_mesh,
    scratch_types=[
        pltpu.VMEM((NCHUNK, K), jnp.int32),
        pltpu.VMEM((NCHUNK, K), jnp.int32),
        pltpu.VMEM((K,), jnp.float32),
        pltpu.VMEM((RB,), jnp.float32),
        pltpu.VMEM((N,), jnp.float32),
        pltpu.VMEM_SHARED((N,), jnp.float32),
        pltpu.VMEM_SHARED((N,), jnp.float32),
        pltpu.SemaphoreType.DMA,
        pltpu.SemaphoreType.DMA,
    ],
)
def _sc_counts(ii_hbm, out_hbm, i0_v, i1_v, ones_v, zb_v, stage_v,
               c0_sh, c1_sh, csem0, csem1):
    c = lax.axis_index("c")
    s = lax.axis_index("s")
    wid = c * NS + s

    for j in range(K // 16):
        ones_v[pl.ds(j * 16, 16)] = jnp.ones((16,), jnp.float32)

    @pl.loop(0, RB // 16)
    def _zf(i):
        zb_v[pl.ds(i * 16, 16)] = _zeros16()

    start = _row_start(s)
    pltpu.sync_copy(zb_v, c0_sh.at[pl.ds(start, RB)])
    pltpu.sync_copy(zb_v, c1_sh.at[pl.ds(start, RB)])
    pltpu.sync_copy(ii_hbm.at[0, wid], i0_v)
    pltpu.sync_copy(ii_hbm.at[1, wid], i1_v)

    plsc.subcore_barrier()

    @pl.loop(0, NCHUNK)
    def _body(g):
        pltpu.async_copy(ones_v, c0_sh.at[i0_v.at[g]], csem0, add=True)
        pltpu.async_copy(ones_v, c1_sh.at[i1_v.at[g]], csem1, add=True)

    pltpu.make_async_copy(out_hbm.at[pl.ds(0, N)], stage_v, csem0).wait()
    pltpu.make_async_copy(out_hbm.at[pl.ds(0, N)], stage_v, csem1).wait()

    plsc.subcore_barrier()

    @pl.when(s == 0)
    def _drain():
        pltpu.sync_copy(c0_sh, stage_v)
        pltpu.sync_copy(stage_v, out_hbm.at[pl.ds((2 * c) * N, N)])
        pltpu.sync_copy(c1_sh, stage_v)
        pltpu.sync_copy(stage_v, out_hbm.at[pl.ds((2 * c + 1) * N, N)])


def _make_seg(gr, sr):
    @functools.partial(
        pl.kernel,
        out_type=jax.ShapeDtypeStruct((2 * N, D), jnp.float32),
        mesh=_mesh,
        scratch_types=[
            pltpu.VMEM((NCHUNK, K), jnp.int32),
            pltpu.VMEM((K,), jnp.int32),
            pltpu.VMEM((K,), jnp.int32),
            pltpu.VMEM((K, D), jnp.float32),
            pltpu.VMEM((K, D), jnp.float32),
            pltpu.VMEM_SHARED((N, D), jnp.float32),
            pltpu.SemaphoreType.DMA,
            pltpu.SemaphoreType.DMA,
            pltpu.SemaphoreType.DMA,
            pltpu.SemaphoreType.DMA,
        ],
    )
    def seg(table_hbm, ii_hbm, out_hbm, si_v, gi0_v, gi1_v,
            rows0_v, rows1_v, acc_sh, sem0, sem1, isem0, isem1):
        c = lax.axis_index("c")
        s = lax.axis_index("s")
        wid = c * NS + s

        @pl.loop(0, K)
        def _zf(i):
            for j in range(D // 16):
                rows1_v[i, pl.ds(j * 16, 16)] = _zeros16()

        start = _row_start(s)
        for t in range(8):
            pltpu.async_copy(rows1_v, acc_sh.at[pl.ds(start + t * K, K)], isem1)

        pltpu.sync_copy(ii_hbm.at[sr, wid], si_v)
        pltpu.sync_copy(ii_hbm.at[gr, wid, 0], gi0_v)
        pltpu.sync_copy(ii_hbm.at[gr, wid, 1], gi1_v)
        pltpu.async_copy(table_hbm.at[gi0_v], rows0_v, sem0)

        for t in range(8):
            pltpu.make_async_copy(rows1_v, acc_sh.at[pl.ds(start, K)], isem1).wait()
        pltpu.async_copy(table_hbm.at[gi1_v], rows1_v, sem1)

        plsc.subcore_barrier()

        @pl.loop(0, NCHUNK - 1, step=2)
        def _body(g):
            pltpu.make_async_copy(table_hbm.at[gi0_v], rows0_v, sem0).wait()
            pltpu.async_copy(ii_hbm.at[gr, wid, g + 2], gi0_v, isem0)
            pltpu.sync_copy(rows0_v, acc_sh.at[si_v.at[g]], add=True)
            pltpu.make_async_copy(ii_hbm.at[gr, wid, 0], gi0_v, isem0).wait()
            pltpu.async_copy(table_hbm.at[gi0_v], rows0_v, sem0)
            pltpu.make_async_copy(table_hbm.at[gi1_v], rows1_v, sem1).wait()

            @pl.when(g + 3 < NCHUNK)
            def _pf():
                pltpu.async_copy(ii_hbm.at[gr, wid, g + 3], gi1_v, isem1)

            pltpu.sync_copy(rows1_v, acc_sh.at[si_v.at[g + 1]], add=True)

            @pl.when(g + 3 < NCHUNK)
            def _g3():
                pltpu.make_async_copy(ii_hbm.at[gr, wid, 0], gi1_v, isem1).wait()
                pltpu.async_copy(table_hbm.at[gi1_v], rows1_v, sem1)

        pltpu.make_async_copy(table_hbm.at[gi0_v], rows0_v, sem0).wait()
        pltpu.sync_copy(rows0_v, acc_sh.at[si_v.at[NCHUNK - 1]], add=True)

        plsc.subcore_barrier()
        pltpu.sync_copy(
            acc_sh.at[pl.ds(start, RB)],
            out_hbm.at[pl.ds(c * N + start, RB)],
        )

    return seg


_seg_a = _make_seg(0, 1)
_seg_b = _make_seg(1, 0)


BLK = 1000


def _mm_body(x_ref, w_ref, o_ref):
    o_ref[...] = jnp.dot(x_ref[...], w_ref[...], preferred_element_type=jnp.float32)


def _tc_mm(x, w):
    return pl.pallas_call(
        _mm_body,
        grid=(N // BLK,),
        in_specs=[
            pl.BlockSpec((BLK, D), lambda i: (i, 0)),
            pl.BlockSpec((D, D), lambda i: (0, 0)),
        ],
        out_specs=pl.BlockSpec((BLK, D), lambda i: (i, 0)),
        out_shape=jax.ShapeDtypeStruct((N, D), jnp.float32),
    )(x, w)


def _recips_body(c_ref, o_ref):
    tot = c_ref[0:1, :] + c_ref[1:2, :]
    o_ref[...] = jnp.where(tot > 0, 1.0 / tot, 0.0)


def _tc_recips(cnts):
    return pl.pallas_call(
        _recips_body,
        out_shape=jax.ShapeDtypeStruct((1, 2 * N), jnp.float32),
    )(cnts)


def _comb_e_body(p_ref, r_ref, o_ref):
    o_ref[...] = (p_ref[0] + p_ref[1]) * r_ref[...]


def _tc_comb_e(p, r):
    return pl.pallas_call(
        _comb_e_body,
        grid=(N // BLK,),
        in_specs=[
            pl.BlockSpec((2, BLK, D), lambda i: (0, i, 0)),
            pl.BlockSpec((BLK, 1), lambda i: (i, 0)),
        ],
        out_specs=pl.BlockSpec((BLK, D), lambda i: (i, 0)),
        out_shape=jax.ShapeDtypeStruct((N, D), jnp.float32),
    )(p, r)


def _hmm_body(q_ref, r_ref, b_ref, w_ref, o_ref):
    h = (q_ref[0] + q_ref[1]) * r_ref[...] + b_ref[...]
    h = jnp.where(h > 0, h, jnp.exp(h) - 1.0)
    o_ref[...] = jnp.dot(h, w_ref[...], preferred_element_type=jnp.float32)


def _tc_hmm(q, r, b, w):
    return pl.pallas_call(
        _hmm_body,
        grid=(N // BLK,),
        in_specs=[
            pl.BlockSpec((2, BLK, D), lambda i: (0, i, 0)),
            pl.BlockSpec((BLK, 1), lambda i: (i, 0)),
            pl.BlockSpec((1, D), lambda i: (0, 0)),
            pl.BlockSpec((D, D), lambda i: (0, 0)),
        ],
        out_specs=pl.BlockSpec((BLK, D), lambda i: (i, 0)),
        out_shape=jax.ShapeDtypeStruct((N, D), jnp.float32),
    )(q, r, b, w)


def _fin_body(q_ref, r_ref, b_ref, o_ref):
    o_ref[...] = (q_ref[0] + q_ref[1]) * r_ref[...] + b_ref[...]


def _tc_final(q, r, b):
    return pl.pallas_call(
        _fin_body,
        grid=(N // BLK,),
        in_specs=[
            pl.BlockSpec((2, BLK, D), lambda i: (0, i, 0)),
            pl.BlockSpec((BLK, 1), lambda i: (i, 0)),
            pl.BlockSpec((1, D), lambda i: (0, 0)),
        ],
        out_specs=pl.BlockSpec((BLK, D), lambda i: (i, 0)),
        out_shape=jax.ShapeDtypeStruct((N, D), jnp.float32),
    )(q, r, b)


def kernel(x, hyperedge_index, W1, b1, W2, b2):
    ii = hyperedge_index.reshape(2, NW, NCHUNK, K)

    cnts = _sc_counts(ii).reshape(2, 2 * N)
    recips = _tc_recips(cnts).reshape(2, N)
    rD = recips[0].reshape(N, 1)
    rB = recips[1].reshape(N, 1)

    b1r = b1.reshape(1, D)
    b2r = b2.reshape(1, D)

    xw1 = _tc_mm(x, W1)
    p1 = _seg_a(xw1, ii).reshape(2, N, D)
    e1 = _tc_comb_e(p1, rB)
    q1 = _seg_b(e1, ii).reshape(2, N, D)
    xw2 = _tc_hmm(q1, rD, b1r, W2)
    p2 = _seg_a(xw2, ii).reshape(2, N, D)
    e2 = _tc_comb_e(p2, rB)
    q2 = _seg_b(e2, ii).reshape(2, N, D)
    out = _tc_final(q2, rD, b2r)
    return (out, e2)
